# baseline (speedup 1.0000x reference)
import functools

import jax
import jax.numpy as jnp
from jax import lax
from jax.experimental import pallas as pl
from jax.experimental.pallas import tpu as pltpu
from jax.experimental.pallas import tpu_sc as plsc

LANES = 16
NWORK = 32
RPC = 5
CHUNK = 512
KBITS = 32


def _make_addr_body(n_neu, entry, tup, n_words):
    def addr_body(tm_ref, x_ref, mem_ref, o_ref, packed_ref, w_ref):
        @pl.when(pl.program_id(1) == 0)
        def _():
            iota = lax.broadcasted_iota(jnp.int32, (n_neu, entry), 1)
            acc = jnp.zeros((n_neu, entry), jnp.float32)
            for t in range(tup):
                wt = jnp.float32(2.0 ** (tup - 1 - t))
                acc = jnp.where(tm_ref[0, :, t:t + 1] == iota, wt, acc)
            w_ref[...] = acc.astype(jnp.bfloat16)

            bits = mem_ref[0].astype(jnp.int32)
            pk = jnp.zeros((n_neu, n_words), jnp.int32)
            for k in range(KBITS):
                pk = pk + (bits[:, k * n_words:(k + 1) * n_words] << k)
            packed_ref[...] = pk[None]

        addr = lax.dot_general(
            w_ref[...], x_ref[...].astype(jnp.bfloat16),
            (((1,), (1,)), ((), ())),
            preferred_element_type=jnp.float32).astype(jnp.int32)
        o_ref[...] = addr[None]

    return addr_body


def _reduce_body(*refs):
    o_ref = refs[-1]
    off = 0
    for p_ref in refs[:-1]:
        h = p_ref.shape[1]
        o_ref[pl.ds(off, h)] = jnp.sum(p_ref[...], axis=0)
        off += h


def _make_sc_gather(n_cls, n_neu, n_words, batch):
    wshift = n_words.bit_length() - 1
    nslot = n_cls * RPC
    nch = batch // CHUNK
    mesh = plsc.VectorSubcoreMesh(core_axis_name="c", subcore_axis_name="s")

    @functools.partial(
        pl.kernel,
        out_type=jax.ShapeDtypeStruct((NWORK, n_cls * batch), jnp.float32),
        mesh=mesh,
        compiler_params=pltpu.CompilerParams(needs_layout_passes=False),
        scratch_types=[
            pltpu.VMEM((nslot * n_words,), jnp.int32),
            pltpu.VMEM((nslot * CHUNK,), jnp.int32),
            pltpu.VMEM((nslot * CHUNK,), jnp.int32),
            pltpu.VMEM((n_cls * batch,), jnp.float32),
            pltpu.SemaphoreType.DMA,
            pltpu.SemaphoreType.DMA,
            pltpu.SemaphoreType.DMA,
        ],
    )
    def sc_gather(packed_hbm, addr_hbm, out_hbm, pbuf, abuf0, abuf1, acc_v,
                  sp, sa0, sa1):
        wid = lax.axis_index("s") * 2 + lax.axis_index("c")
        valid5 = wid < (n_neu - (RPC - 1) * NWORK)

        def for_slots(fn):
            for c in range(n_cls):
                for mm in range(RPC):
                    s = c * RPC + mm
                    n = wid + mm * NWORK
                    if mm < RPC - 1:
                        fn(c, mm, s, n)
                    else:
                        pl.when(valid5)(lambda c=c, mm=mm, s=s, n=n:
                                        fn(c, mm, s, n))

        def issue_packed(c, mm, s, n):
            pltpu.async_copy(packed_hbm.at[c, n],
                             pbuf.at[pl.ds(s * n_words, n_words)], sp)

        def wait_packed(c, mm, s, n):
            pltpu.make_async_copy(packed_hbm.at[c, n],
                                  pbuf.at[pl.ds(s * n_words, n_words)],
                                  sp).wait()

        def make_issue_addr(ch, ab, sa):
            def issue_addr(c, mm, s, n):
                pltpu.async_copy(addr_hbm.at[c, n, pl.ds(ch * CHUNK, CHUNK)],
                                 ab.at[pl.ds(s * CHUNK, CHUNK)], sa)
            return issue_addr

        def make_wait_addr(ab, sa):
            def wait_addr(c, mm, s, n):
                pltpu.make_async_copy(addr_hbm.at[c, n, pl.ds(0, CHUNK)],
                                      ab.at[pl.ds(s * CHUNK, CHUNK)],
                                      sa).wait()
            return wait_addr

        def compute(ch, ab):
            def gbody(g, _):
                for c in range(n_cls):
                    acc = None
                    for mm in range(RPC):
                        s = c * RPC + mm
                        idx = ab[pl.ds(s * CHUNK + g * LANES, LANES)]
                        gi = (idx & (n_words - 1)) + (s * n_words)
                        word = plsc.load_gather(pbuf, [gi])
                        bit = (word >> (idx >> wshift)) & 1
                        if mm == RPC - 1:
                            bit = jnp.where(valid5, bit, 0)
                        acc = bit if acc is None else acc + bit
                    dst = pl.ds(c * batch + ch * CHUNK + g * LANES, LANES)
                    acc_v[dst] = acc.astype(jnp.float32)
                return 0

            lax.fori_loop(0, CHUNK // LANES, gbody, 0, unroll=2)

        for_slots(issue_packed)
        for_slots(make_issue_addr(0, abuf0, sa0))
        for_slots(wait_packed)

        def pairbody(p, _):
            ch0 = 2 * p
            ch1 = ch0 + 1
            for_slots(make_wait_addr(abuf0, sa0))
            for_slots(make_issue_addr(ch1, abuf1, sa1))
            compute(ch0, abuf0)
            for_slots(make_wait_addr(abuf1, sa1))

            @pl.when(ch0 + 2 < nch)
            def _():
                for_slots(make_issue_addr(ch0 + 2, abuf0, sa0))

            compute(ch1, abuf1)
            return 0

        lax.fori_loop(0, nch // 2, pairbody, 0)

        pltpu.sync_copy(acc_v, out_hbm.at[wid])

    return sc_gather


def kernel(samples, tuple_mapping, memory):
    n_cls, n_neu, n_addr = memory.shape
    batch, entry = samples.shape
    total = tuple_mapping.shape[1]
    tup = total // n_neu
    n_words = n_addr // KBITS

    tm3 = tuple_mapping.reshape(n_cls, n_neu, tup)

    parts = (5, 5)
    nbq = 4
    bq = batch // nbq
    sc_calls = {s: _make_sc_gather(s, n_neu, n_words, batch)
                for s in set(parts)}

    def tc_part(start, size):
        return pl.pallas_call(
            _make_addr_body(n_neu, entry, tup, n_words),
            grid=(size, nbq),
            in_specs=[
                pl.BlockSpec((1, n_neu, tup), lambda i, j: (i + start, 0, 0)),
                pl.BlockSpec((bq, entry), lambda i, j: (j, 0)),
                pl.BlockSpec((1, n_neu, n_addr),
                             lambda i, j: (i + start, 0, 0)),
            ],
            out_specs=[
                pl.BlockSpec((1, n_neu, bq), lambda i, j: (i, 0, j)),
                pl.BlockSpec((1, n_neu, n_words), lambda i, j: (i, 0, 0)),
            ],
            out_shape=[
                jax.ShapeDtypeStruct((size, n_neu, batch), jnp.int32),
                jax.ShapeDtypeStruct((size, n_neu, n_words), jnp.int32),
            ],
            scratch_shapes=[pltpu.VMEM((n_neu, entry), jnp.bfloat16)],
        )(tm3, samples, memory)

    partials = []
    start = 0
    for size in parts:
        addr_p, packed_p = tc_part(start, size)
        partials.append(sc_calls[size](packed_p, addr_p))
        start += size

    resp = pl.pallas_call(
        _reduce_body,
        out_shape=jax.ShapeDtypeStruct((n_cls * batch,), jnp.float32),
    )(*partials)
    return resp.reshape(n_cls, batch).T


# speedup vs baseline: 1.3070x; 1.3070x over previous
import functools

import jax
import jax.numpy as jnp
from jax import lax
from jax.experimental import pallas as pl
from jax.experimental.pallas import tpu as pltpu
from jax.experimental.pallas import tpu_sc as plsc

LANES = 16
NWORK = 32
RPC = 5
CHUNK = 512
KBITS = 32


def _make_addr_body(n_neu, entry, tup, n_words):
    def addr_body(tm_ref, x_ref, mem_ref, o_ref, packed_ref, w_ref):
        iota = lax.broadcasted_iota(jnp.int32, (n_neu, entry), 1)
        acc = jnp.zeros((n_neu, entry), jnp.float32)
        for t in range(tup):
            wt = jnp.float32(2.0 ** (tup - 1 - t))
            acc = jnp.where(tm_ref[0, :, t:t + 1] == iota, wt, acc)
        w_ref[...] = acc.astype(jnp.bfloat16)

        addr = lax.dot_general(
            w_ref[...], x_ref[...], (((1,), (1,)), ((), ())),
            preferred_element_type=jnp.float32).astype(jnp.int32)
        o_ref[...] = addr[None]

        bits = mem_ref[0].astype(jnp.int32)
        pk = jnp.zeros((n_neu, n_words), jnp.int32)
        for k in range(KBITS):
            pk = pk + (bits[:, k * n_words:(k + 1) * n_words] << k)
        packed_ref[...] = pk[None]

    return addr_body


def _reduce_body(*refs):
    o_ref = refs[-1]
    off = 0
    for p_ref in refs[:-1]:
        h = p_ref.shape[1]
        o_ref[pl.ds(off, h)] = jnp.sum(p_ref[...], axis=0)
        off += h


def _make_sc_gather(n_cls, n_neu, n_words, batch):
    wshift = n_words.bit_length() - 1
    nslot = n_cls * RPC
    nch = batch // CHUNK
    mesh = plsc.VectorSubcoreMesh(core_axis_name="c", subcore_axis_name="s")

    @functools.partial(
        pl.kernel,
        out_type=jax.ShapeDtypeStruct((NWORK, n_cls * batch), jnp.float32),
        mesh=mesh,
        compiler_params=pltpu.CompilerParams(needs_layout_passes=False),
        scratch_types=[
            pltpu.VMEM((nslot * n_words,), jnp.int32),
            pltpu.VMEM((nslot * CHUNK,), jnp.int32),
            pltpu.VMEM((nslot * CHUNK,), jnp.int32),
            pltpu.VMEM((n_cls * batch,), jnp.float32),
            pltpu.SemaphoreType.DMA,
            pltpu.SemaphoreType.DMA,
            pltpu.SemaphoreType.DMA,
        ],
    )
    def sc_gather(packed_hbm, addr_hbm, out_hbm, pbuf, abuf0, abuf1, acc_v,
                  sp, sa0, sa1):
        wid = lax.axis_index("s") * 2 + lax.axis_index("c")
        valid5 = wid < (n_neu - (RPC - 1) * NWORK)

        def for_slots(fn):
            for c in range(n_cls):
                for mm in range(RPC):
                    s = c * RPC + mm
                    n = wid + mm * NWORK
                    if mm < RPC - 1:
                        fn(c, mm, s, n)
                    else:
                        pl.when(valid5)(lambda c=c, mm=mm, s=s, n=n:
                                        fn(c, mm, s, n))

        def issue_packed(c, mm, s, n):
            pltpu.async_copy(packed_hbm.at[c, n],
                             pbuf.at[pl.ds(s * n_words, n_words)], sp)

        def wait_packed(c, mm, s, n):
            pltpu.make_async_copy(packed_hbm.at[c, n],
                                  pbuf.at[pl.ds(s * n_words, n_words)],
                                  sp).wait()

        def make_issue_addr(ch, ab, sa):
            def issue_addr(c, mm, s, n):
                pltpu.async_copy(addr_hbm.at[c, n, pl.ds(ch * CHUNK, CHUNK)],
                                 ab.at[pl.ds(s * CHUNK, CHUNK)], sa)
            return issue_addr

        def make_wait_addr(ab, sa):
            def wait_addr(c, mm, s, n):
                pltpu.make_async_copy(addr_hbm.at[c, n, pl.ds(0, CHUNK)],
                                      ab.at[pl.ds(s * CHUNK, CHUNK)],
                                      sa).wait()
            return wait_addr

        def compute(ch, ab):
            def gbody(g, _):
                for c in range(n_cls):
                    acc = None
                    for mm in range(RPC):
                        s = c * RPC + mm
                        idx = ab[pl.ds(s * CHUNK + g * LANES, LANES)]
                        gi = (idx & (n_words - 1)) + (s * n_words)
                        word = plsc.load_gather(pbuf, [gi])
                        bit = (word >> (idx >> wshift)) & 1
                        if mm == RPC - 1:
                            bit = jnp.where(valid5, bit, 0)
                        acc = bit if acc is None else acc + bit
                    dst = pl.ds(c * batch + ch * CHUNK + g * LANES, LANES)
                    acc_v[dst] = acc.astype(jnp.float32)
                return 0

            lax.fori_loop(0, CHUNK // LANES, gbody, 0, unroll=2)

        for_slots(issue_packed)
        for_slots(make_issue_addr(0, abuf0, sa0))
        for_slots(wait_packed)

        def pairbody(p, _):
            ch0 = 2 * p
            ch1 = ch0 + 1
            for_slots(make_wait_addr(abuf0, sa0))
            for_slots(make_issue_addr(ch1, abuf1, sa1))
            compute(ch0, abuf0)
            for_slots(make_wait_addr(abuf1, sa1))

            @pl.when(ch0 + 2 < nch)
            def _():
                for_slots(make_issue_addr(ch0 + 2, abuf0, sa0))

            compute(ch1, abuf1)
            return 0

        lax.fori_loop(0, nch // 2, pairbody, 0)

        pltpu.sync_copy(acc_v, out_hbm.at[wid])

    return sc_gather


def kernel(samples, tuple_mapping, memory):
    n_cls, n_neu, n_addr = memory.shape
    batch, entry = samples.shape
    total = tuple_mapping.shape[1]
    tup = total // n_neu
    n_words = n_addr // KBITS

    tm3 = tuple_mapping.reshape(n_cls, n_neu, tup)
    xb = samples.astype(jnp.bfloat16)

    parts = (5, 5)
    sc_calls = {s: _make_sc_gather(s, n_neu, n_words, batch)
                for s in set(parts)}

    def tc_part(start, size):
        return pl.pallas_call(
            _make_addr_body(n_neu, entry, tup, n_words),
            grid=(size,),
            in_specs=[
                pl.BlockSpec((1, n_neu, tup), lambda i: (i + start, 0, 0)),
                pl.BlockSpec((batch, entry), lambda i: (0, 0)),
                pl.BlockSpec((1, n_neu, n_addr), lambda i: (i + start, 0, 0)),
            ],
            out_specs=[
                pl.BlockSpec((1, n_neu, batch), lambda i: (i, 0, 0)),
                pl.BlockSpec((1, n_neu, n_words), lambda i: (i, 0, 0)),
            ],
            out_shape=[
                jax.ShapeDtypeStruct((size, n_neu, batch), jnp.int32),
                jax.ShapeDtypeStruct((size, n_neu, n_words), jnp.int32),
            ],
            scratch_shapes=[pltpu.VMEM((n_neu, entry), jnp.bfloat16)],
        )(tm3, xb, memory)

    partials = []
    start = 0
    for size in parts:
        addr_p, packed_p = tc_part(start, size)
        partials.append(sc_calls[size](packed_p, addr_p))
        start += size

    resp = pl.pallas_call(
        _reduce_body,
        out_shape=jax.ShapeDtypeStruct((n_cls * batch,), jnp.float32),
    )(*partials)
    return resp.reshape(n_cls, batch).T
